# baseline (speedup 1.0000x reference)
import functools

import jax
import jax.numpy as jnp
from jax import lax
from jax.experimental import pallas as pl
from jax.experimental.pallas import tpu as pltpu
from jax.experimental.pallas import tpu_sc as plsc

_T = 4096
_D = 2048
_R = 512
_FF = 768
_E = 8
_K = 2
_B = 256
_P = _T * _K + _E * _B
_NB = _P // _B
_NW = 32
_GCH = 64
_GNB = 5
_CCH = 8
_CNB = 3
_CNC = (_T // _NW) // _CCH


def _lowrank_proj_kernel(x_ref, vh_ref, out_ref):
    out_ref[...] = lax.dot_general(
        x_ref[...].astype(jnp.bfloat16), vh_ref[...], (((1,), (1,)), ((), ())),
        preferred_element_type=jnp.float32).astype(jnp.bfloat16)


def _expert_block_kernel(be_ref, glow_ref, u_ref, d_ref, w_ref, y_ref):
    gu = lax.dot_general(glow_ref[...], u_ref[0],
                         (((1,), (1,)), ((), ())),
                         preferred_element_type=jnp.float32)
    gate = gu[:, :_FF]
    up = gu[:, _FF:]
    h = gate * jax.nn.sigmoid(gate) * up
    y = lax.dot_general(h.astype(jnp.bfloat16), d_ref[0],
                        (((1,), (1,)), ((), ())),
                        preferred_element_type=jnp.float32)
    y_ref[...] = y * w_ref[0, 0][:, None]


def _sc_gather(low2, gidx):
    mesh = plsc.VectorSubcoreMesh(core_axis_name="c", subcore_axis_name="s")
    rows_per_t = _P // 16
    nch = rows_per_t // _GCH
    hc = _R // 4
    stage_rows = (2 * _T) // 16

    @functools.partial(
        pl.kernel, mesh=mesh,
        out_type=jax.ShapeDtypeStruct((_P, _R // 2), jnp.int32),
        scratch_types=(
            [pltpu.VMEM_SHARED((2 * _T, hc), jnp.int32)]
            + [pltpu.VMEM((rows_per_t,), jnp.int32)]
            + [pltpu.VMEM((_GCH, hc), jnp.int32) for _ in range(_GNB)]
            + [pltpu.SemaphoreType.DMA for _ in range(2 * _GNB)]
        ),
    )
    def k(low2_hbm, gidx_hbm, glow_hbm, *refs):
        tab = refs[0]
        idx = refs[1]
        bufs = refs[2:2 + _GNB]
        gs = refs[2 + _GNB:2 + 2 * _GNB]
        ws = refs[2 + 2 * _GNB:2 + 3 * _GNB]
        cid = lax.axis_index("c")
        sid = lax.axis_index("s")
        col = cid * hc
        base = sid * rows_per_t
        pltpu.sync_copy(
            low2_hbm.at[pl.ds(sid * stage_rows, stage_rows), pl.ds(col, hc)],
            tab.at[pl.ds(sid * stage_rows, stage_rows)])
        pltpu.sync_copy(gidx_hbm.at[pl.ds(base, rows_per_t)], idx)
        plsc.subcore_barrier()

        def fire(c):
            b = c % _GNB
            return pltpu.async_copy(
                tab.at[idx.at[pl.ds(c * _GCH, _GCH)]], bufs[b], gs[b])

        gcp = [None] * nch
        wcp = [None] * nch
        for c in range(min(_GNB, nch)):
            gcp[c] = fire(c)
        for c in range(nch):
            b = c % _GNB
            gcp[c].wait()
            wcp[c] = pltpu.async_copy(
                bufs[b],
                glow_hbm.at[pl.ds(base + c * _GCH, _GCH), pl.ds(col, hc)],
                ws[b])
            if c + _GNB < nch:
                wcp[c].wait()
                gcp[c + _GNB] = fire(c + _GNB)
        for c in range(max(0, nch - _GNB), nch):
            wcp[c].wait()

    return k(low2, gidx)


def _sc_combine(y, pos0, pos1):
    mesh = plsc.VectorSubcoreMesh(core_axis_name="c", subcore_axis_name="s")
    tok_per_w = _T // _NW

    @functools.partial(
        pl.kernel, mesh=mesh,
        out_type=jax.ShapeDtypeStruct((_T, _D), jnp.float32),
        scratch_types=(
            [pltpu.VMEM((tok_per_w,), jnp.int32) for _ in range(2)]
            + [pltpu.VMEM((_CCH, _D), jnp.float32) for _ in range(2 * _CNB)]
            + [pltpu.SemaphoreType.DMA for _ in range(3 * _CNB)]
        ),
    )
    def k(y_hbm, p0_hbm, p1_hbm, out_hbm, *refs):
        i0, i1 = refs[0], refs[1]
        av = refs[2:2 + _CNB]
        bv = refs[2 + _CNB:2 + 2 * _CNB]
        gsa = refs[2 + 2 * _CNB:2 + 3 * _CNB]
        gsb = refs[2 + 3 * _CNB:2 + 4 * _CNB]
        wsm = refs[2 + 4 * _CNB:2 + 5 * _CNB]
        wid = lax.axis_index("s") * 2 + lax.axis_index("c")
        base = wid * tok_per_w
        pltpu.sync_copy(p0_hbm.at[pl.ds(base, tok_per_w)], i0)
        pltpu.sync_copy(p1_hbm.at[pl.ds(base, tok_per_w)], i1)

        def fire(c):
            s = c % _CNB
            o = pl.ds(c * _CCH, _CCH)
            ga = pltpu.async_copy(y_hbm.at[i0.at[o]], av[s], gsa[s])
            gb = pltpu.async_copy(y_hbm.at[i1.at[o]], bv[s], gsb[s])
            return ga, gb

        gcp = [None] * _CNC
        wcp = [None] * _CNC
        for c in range(min(_CNB, _CNC)):
            gcp[c] = fire(c)
        for c in range(_CNC):
            s = c % _CNB
            gcp[c][0].wait()
            gcp[c][1].wait()
            for r in range(_CCH):
                def body(j, _, r=r, s=s):
                    sl = pl.ds(j * 16, 16)
                    av[s][r, sl] = av[s][r, sl] + bv[s][r, sl]
                    return 0
                lax.fori_loop(0, _D // 16, body, 0, unroll=8)
            wcp[c] = pltpu.async_copy(
                av[s], out_hbm.at[pl.ds(base + c * _CCH, _CCH)], wsm[s])
            if c + _CNB < _CNC:
                wcp[c].wait()
                gcp[c + _CNB] = fire(c + _CNB)
        for c in range(max(0, _CNC - _CNB), _CNC):
            wcp[c].wait()

    return k(y, pos0, pos1)


def _routing(idx, w):
    tk = _T * _K
    flat_e = idx.reshape(-1)
    flat_w = w.reshape(-1)
    tok_of_pair = jnp.arange(tk, dtype=jnp.int32) // _K
    order = jnp.argsort(flat_e)
    inv = jnp.argsort(order).astype(jnp.int32)
    counts = jnp.bincount(flat_e, length=_E).astype(jnp.int32)
    off = jnp.concatenate(
        [jnp.zeros((1,), jnp.int32), jnp.cumsum(counts)[:-1].astype(jnp.int32)])
    padc = ((counts + _B - 1) // _B) * _B
    pad_off = jnp.concatenate(
        [jnp.zeros((1,), jnp.int32), jnp.cumsum(padc)[:-1].astype(jnp.int32)])
    starts = jnp.arange(_NB, dtype=jnp.int32) * _B
    be = (jnp.searchsorted(pad_off, starts, side='right') - 1).astype(jnp.int32)
    slot_e = jnp.repeat(be, _B)
    slot_i = jnp.arange(_P, dtype=jnp.int32)
    r_un = slot_i - pad_off[slot_e] + off[slot_e]
    valid = (slot_i - pad_off[slot_e]) < counts[slot_e]
    pair = order[jnp.clip(r_un, 0, tk - 1)]
    slot_tok = jnp.where(valid, tok_of_pair[pair], 0)
    slot_w = jnp.where(valid, flat_w[pair], 0.0)
    gidx = slot_tok * 2 + (slot_e // (_E // 2))
    pos = (pad_off[flat_e] + (inv - off[flat_e])).reshape(_T, _K)
    return be, slot_w, gidx, pos


def kernel(hidden_states, top_k_weights, vh0, vh1, u0, u1, d0, d1, top_k_index):
    idx = top_k_index.astype(jnp.int32)
    be, slot_w, gidx, pos = _routing(idx, top_k_weights)

    vhcat = jnp.concatenate([vh0, vh1], axis=0).astype(jnp.bfloat16)
    tb = 512
    low = pl.pallas_call(
        _lowrank_proj_kernel,
        grid=(_T // tb,),
        in_specs=[pl.BlockSpec((tb, _D), lambda i: (i, 0)),
                  pl.BlockSpec((2 * _R, _D), lambda i: (0, 0))],
        out_specs=pl.BlockSpec((tb, 2 * _R), lambda i: (i, 0)),
        out_shape=jax.ShapeDtypeStruct((_T, 2 * _R), jnp.bfloat16),
    )(hidden_states, vhcat)
    low2 = low.reshape(_T * 2, _R)

    low2i = lax.bitcast_convert_type(
        low2.reshape(_T * 2, _R // 2, 2), jnp.int32)
    glowi = _sc_gather(low2i, gidx)
    glow = lax.bitcast_convert_type(glowi, jnp.bfloat16).reshape(_P, _R)

    u_all = jnp.concatenate([u0, u1], axis=0).astype(jnp.bfloat16)
    d_all = jnp.concatenate([d0, d1], axis=0).astype(jnp.bfloat16)
    w3 = slot_w.reshape(_NB, 1, _B)
    grid_spec = pltpu.PrefetchScalarGridSpec(
        num_scalar_prefetch=1,
        grid=(_NB,),
        in_specs=[
            pl.BlockSpec((_B, _R), lambda b, be_ref: (b, 0)),
            pl.BlockSpec((1, 2 * _FF, _R), lambda b, be_ref: (be_ref[b], 0, 0)),
            pl.BlockSpec((1, _D, _FF), lambda b, be_ref: (be_ref[b], 0, 0)),
            pl.BlockSpec((1, 1, _B), lambda b, be_ref: (b, 0, 0)),
        ],
        out_specs=pl.BlockSpec((_B, _D), lambda b, be_ref: (b, 0)),
    )
    y = pl.pallas_call(
        _expert_block_kernel,
        grid_spec=grid_spec,
        out_shape=jax.ShapeDtypeStruct((_P, _D), jnp.float32),
    )(be, glow, u_all, d_all, w3)

    return _sc_combine(y, pos[:, 0], pos[:, 1])


# speedup vs baseline: 4.7482x; 4.7482x over previous
import functools

import jax
import jax.numpy as jnp
from jax import lax
from jax.experimental import pallas as pl
from jax.experimental.pallas import tpu as pltpu
from jax.experimental.pallas import tpu_sc as plsc

_T = 4096
_D = 2048
_R = 512
_FF = 768
_E = 8
_K = 2
_B = 256
_P = _T * _K + _E * _B
_NB = _P // _B
_NW = 32
_GCH = 64
_GNB = 5
_CCH = 8
_CNB = 3
_CNC = (_T // _NW) // _CCH


def _lowrank_proj_kernel(x_ref, vh_ref, out_ref):
    low = lax.dot_general(
        x_ref[...].astype(jnp.bfloat16), vh_ref[...], (((1,), (1,)), ((), ())),
        preferred_element_type=jnp.float32)
    b0 = lax.bitcast_convert_type(
        low[:, :_R].astype(jnp.bfloat16), jnp.uint16).astype(jnp.int32)
    b1 = lax.bitcast_convert_type(
        low[:, _R:].astype(jnp.bfloat16), jnp.uint16).astype(jnp.int32)
    out_ref[...] = b0 | (b1 << 16)


def _expert_block_kernel(be_ref, glow_ref, u_ref, d_ref, w_ref, y_ref):
    w = glow_ref[...]
    g1 = be_ref[pl.program_id(0)] >= (_E // 2)
    fbits = jnp.where(g1, w & jnp.int32(-65536), w << 16)
    glow = lax.bitcast_convert_type(fbits, jnp.float32).astype(jnp.bfloat16)
    gu = lax.dot_general(glow, u_ref[0],
                         (((1,), (1,)), ((), ())),
                         preferred_element_type=jnp.float32)
    gate = gu[:, :_FF]
    up = gu[:, _FF:]
    h = gate * jax.nn.sigmoid(gate) * up
    y = lax.dot_general(h.astype(jnp.bfloat16), d_ref[0],
                        (((1,), (1,)), ((), ())),
                        preferred_element_type=jnp.float32)
    y_ref[...] = y * w_ref[0, 0][:, None]


def _sc_gather(lowp, gidx):
    mesh = plsc.VectorSubcoreMesh(core_axis_name="c", subcore_axis_name="s")
    rows_per_t = _P // 16
    nch = rows_per_t // _GCH
    hc = _R // 4
    stage_rows = _T // 16

    @functools.partial(
        pl.kernel, mesh=mesh,
        out_type=jax.ShapeDtypeStruct((_P, _R), jnp.int32),
        scratch_types=(
            [pltpu.VMEM_SHARED((_T, hc), jnp.int32)]
            + [pltpu.VMEM((rows_per_t,), jnp.int32)]
            + [pltpu.VMEM((_GCH, hc), jnp.int32) for _ in range(_GNB)]
            + [pltpu.SemaphoreType.DMA for _ in range(2 * _GNB)]
        ),
    )
    def k(lowp_hbm, gidx_hbm, glow_hbm, *refs):
        tab = refs[0]
        idx = refs[1]
        bufs = refs[2:2 + _GNB]
        gs = refs[2 + _GNB:2 + 2 * _GNB]
        ws = refs[2 + 2 * _GNB:2 + 3 * _GNB]
        cid = lax.axis_index("c")
        sid = lax.axis_index("s")
        base = sid * rows_per_t
        pltpu.sync_copy(gidx_hbm.at[pl.ds(base, rows_per_t)], idx)
        for half in range(2):
            col = (2 * half) * hc + cid * hc
            pltpu.sync_copy(
                lowp_hbm.at[pl.ds(sid * stage_rows, stage_rows),
                            pl.ds(col, hc)],
                tab.at[pl.ds(sid * stage_rows, stage_rows)])
            plsc.subcore_barrier()

            def fire(c):
                b = c % _GNB
                return pltpu.async_copy(
                    tab.at[idx.at[pl.ds(c * _GCH, _GCH)]], bufs[b], gs[b])

            gcp = [None] * nch
            wcp = [None] * nch
            for c in range(min(_GNB, nch)):
                gcp[c] = fire(c)
            for c in range(nch):
                b = c % _GNB
                gcp[c].wait()
                wcp[c] = pltpu.async_copy(
                    bufs[b],
                    glow_hbm.at[pl.ds(base + c * _GCH, _GCH), pl.ds(col, hc)],
                    ws[b])
                if c + _GNB < nch:
                    wcp[c].wait()
                    gcp[c + _GNB] = fire(c + _GNB)
            for c in range(max(0, nch - _GNB), nch):
                wcp[c].wait()
            plsc.subcore_barrier()

    return k(lowp, gidx)


def _sc_combine(y, pos0, pos1):
    mesh = plsc.VectorSubcoreMesh(core_axis_name="c", subcore_axis_name="s")
    tok_per_w = _T // _NW

    @functools.partial(
        pl.kernel, mesh=mesh,
        out_type=jax.ShapeDtypeStruct((_T, _D), jnp.float32),
        scratch_types=(
            [pltpu.VMEM((tok_per_w,), jnp.int32) for _ in range(2)]
            + [pltpu.VMEM((_CCH, _D), jnp.float32) for _ in range(2 * _CNB)]
            + [pltpu.SemaphoreType.DMA for _ in range(3 * _CNB)]
        ),
    )
    def k(y_hbm, p0_hbm, p1_hbm, out_hbm, *refs):
        i0, i1 = refs[0], refs[1]
        av = refs[2:2 + _CNB]
        bv = refs[2 + _CNB:2 + 2 * _CNB]
        gsa = refs[2 + 2 * _CNB:2 + 3 * _CNB]
        gsb = refs[2 + 3 * _CNB:2 + 4 * _CNB]
        wsm = refs[2 + 4 * _CNB:2 + 5 * _CNB]
        wid = lax.axis_index("s") * 2 + lax.axis_index("c")
        base = wid * tok_per_w
        pltpu.sync_copy(p0_hbm.at[pl.ds(base, tok_per_w)], i0)
        pltpu.sync_copy(p1_hbm.at[pl.ds(base, tok_per_w)], i1)

        def fire(c):
            s = c % _CNB
            o = pl.ds(c * _CCH, _CCH)
            ga = pltpu.async_copy(y_hbm.at[i0.at[o]], av[s], gsa[s])
            gb = pltpu.async_copy(y_hbm.at[i1.at[o]], bv[s], gsb[s])
            return ga, gb

        gcp = [None] * _CNC
        wcp = [None] * _CNC
        for c in range(min(_CNB, _CNC)):
            gcp[c] = fire(c)
        for c in range(_CNC):
            s = c % _CNB
            gcp[c][0].wait()
            gcp[c][1].wait()
            for r in range(_CCH):
                def body(j, _, r=r, s=s):
                    sl = pl.ds(j * 16, 16)
                    av[s][r, sl] = av[s][r, sl] + bv[s][r, sl]
                    return 0
                lax.fori_loop(0, _D // 16, body, 0, unroll=8)
            wcp[c] = pltpu.async_copy(
                av[s], out_hbm.at[pl.ds(base + c * _CCH, _CCH)], wsm[s])
            if c + _CNB < _CNC:
                wcp[c].wait()
                gcp[c + _CNB] = fire(c + _CNB)
        for c in range(max(0, _CNC - _CNB), _CNC):
            wcp[c].wait()

    return k(y, pos0, pos1)


def _routing(idx, w):
    tk = _T * _K
    flat_e = idx.reshape(-1)
    flat_w = w.reshape(-1)
    tok_of_pair = jnp.arange(tk, dtype=jnp.int32) // _K
    order = jnp.argsort(flat_e)
    inv = jnp.argsort(order).astype(jnp.int32)
    counts = jnp.bincount(flat_e, length=_E).astype(jnp.int32)
    off = jnp.concatenate(
        [jnp.zeros((1,), jnp.int32), jnp.cumsum(counts)[:-1].astype(jnp.int32)])
    padc = ((counts + _B - 1) // _B) * _B
    pad_off = jnp.concatenate(
        [jnp.zeros((1,), jnp.int32), jnp.cumsum(padc)[:-1].astype(jnp.int32)])
    starts = jnp.arange(_NB, dtype=jnp.int32) * _B
    be = (jnp.searchsorted(pad_off, starts, side='right') - 1).astype(jnp.int32)
    slot_e = jnp.repeat(be, _B)
    slot_i = jnp.arange(_P, dtype=jnp.int32)
    r_un = slot_i - pad_off[slot_e] + off[slot_e]
    valid = (slot_i - pad_off[slot_e]) < counts[slot_e]
    pair = order[jnp.clip(r_un, 0, tk - 1)]
    slot_tok = jnp.where(valid, tok_of_pair[pair], 0)
    slot_w = jnp.where(valid, flat_w[pair], 0.0)
    gidx = slot_tok
    pos = (pad_off[flat_e] + (inv - off[flat_e])).reshape(_T, _K)
    return be, slot_w, gidx, pos


def kernel(hidden_states, top_k_weights, vh0, vh1, u0, u1, d0, d1, top_k_index):
    idx = top_k_index.astype(jnp.int32)
    be, slot_w, gidx, pos = _routing(idx, top_k_weights)

    vhcat = jnp.concatenate([vh0, vh1], axis=0).astype(jnp.bfloat16)
    tb = 512
    low = pl.pallas_call(
        _lowrank_proj_kernel,
        grid=(_T // tb,),
        in_specs=[pl.BlockSpec((tb, _D), lambda i: (i, 0)),
                  pl.BlockSpec((2 * _R, _D), lambda i: (0, 0))],
        out_specs=pl.BlockSpec((tb, _R), lambda i: (i, 0)),
        out_shape=jax.ShapeDtypeStruct((_T, _R), jnp.int32),
    )(hidden_states, vhcat)

    glow = _sc_gather(low, gidx)

    u_all = jnp.concatenate([u0, u1], axis=0).astype(jnp.bfloat16)
    d_all = jnp.concatenate([d0, d1], axis=0).astype(jnp.bfloat16)
    w3 = slot_w.reshape(_NB, 1, _B)
    grid_spec = pltpu.PrefetchScalarGridSpec(
        num_scalar_prefetch=1,
        grid=(_NB,),
        in_specs=[
            pl.BlockSpec((_B, _R), lambda b, be_ref: (b, 0)),
            pl.BlockSpec((1, 2 * _FF, _R), lambda b, be_ref: (be_ref[b], 0, 0)),
            pl.BlockSpec((1, _D, _FF), lambda b, be_ref: (be_ref[b], 0, 0)),
            pl.BlockSpec((1, 1, _B), lambda b, be_ref: (b, 0, 0)),
        ],
        out_specs=pl.BlockSpec((_B, _D), lambda b, be_ref: (b, 0)),
    )
    y = pl.pallas_call(
        _expert_block_kernel,
        grid_spec=grid_spec,
        out_shape=jax.ShapeDtypeStruct((_P, _D), jnp.float32),
    )(be, glow, u_all, d_all, w3)

    return _sc_combine(y, pos[:, 0], pos[:, 1])
